# baseline (speedup 1.0000x reference)
import functools

import jax
import jax.numpy as jnp
from jax.experimental import pallas as pl
from jax.experimental.pallas import tpu as pltpu


def _se_onepass_kernel(x_ref, w1t_ref, w2_ref, o_ref, slab_ref, acc_ref,
                       *, S, s_tile, inv_s):
    s = pl.program_id(1)
    n_s = pl.num_programs(1)

    @pl.when(s == 0)
    def _():
        acc_ref[...] = jnp.zeros_like(acc_ref)

    x = x_ref[0].astype(jnp.float32)
    slab_ref[:, pl.ds(s * s_tile, s_tile)] = x

    if S % s_tile == 0:
        acc_ref[...] += jnp.sum(x, axis=-1, keepdims=True)
    else:
        @pl.when(s < n_s - 1)
        def _():
            acc_ref[...] += jnp.sum(x, axis=-1, keepdims=True)

        @pl.when(s == n_s - 1)
        def _():
            rem = S - s * s_tile
            lane = jax.lax.broadcasted_iota(jnp.int32, x.shape, 1)
            acc_ref[...] += jnp.sum(
                jnp.where(lane < rem, x, 0.0), axis=-1, keepdims=True)

    @pl.when(s == n_s - 1)
    def _():
        pool = acc_ref[...] * inv_s
        h = jnp.maximum(
            jnp.sum(w1t_ref[...] * pool, axis=0, keepdims=True), 0.0
        )
        gate = jax.nn.sigmoid(
            jnp.sum(w2_ref[...] * h, axis=1, keepdims=True)
        )
        o_ref[0] = (slab_ref[:, :S] * gate).astype(o_ref.dtype)


def _pick_tile(S):
    for t in (2048, 1536, 1024, 2560, 3072):
        if S % t == 0:
            return t
    return min(2048, pl.cdiv(S, 128) * 128)


def kernel(x, w1, w2):
    N, C, D, H, W = x.shape
    hid = w1.shape[0]
    S = D * H * W
    x2 = x.reshape(N, C, S)

    s_tile = _pick_tile(S)
    s_tiles = pl.cdiv(S, s_tile)

    out = pl.pallas_call(
        functools.partial(_se_onepass_kernel, S=S, s_tile=s_tile, inv_s=1.0 / S),
        out_shape=jax.ShapeDtypeStruct((N, C, S), x.dtype),
        grid=(N, s_tiles),
        in_specs=[
            pl.BlockSpec((1, C, s_tile), lambda n, s: (n, 0, s)),
            pl.BlockSpec((C, hid), lambda n, s: (0, 0)),
            pl.BlockSpec((C, hid), lambda n, s: (0, 0)),
        ],
        out_specs=pl.BlockSpec((1, C, S), lambda n, s: (n, 0, 0)),
        scratch_shapes=[
            pltpu.VMEM((C, s_tiles * s_tile), jnp.float32),
            pltpu.VMEM((C, 1), jnp.float32),
        ],
        compiler_params=pltpu.CompilerParams(
            dimension_semantics=("parallel", "arbitrary"),
            vmem_limit_bytes=56 * 1024 * 1024,
        ),
    )(x2, jnp.transpose(w1), w2)

    return out.reshape(N, C, D, H, W)


# speedup vs baseline: 1.0723x; 1.0723x over previous
import functools

import jax
import jax.numpy as jnp
from jax.experimental import pallas as pl
from jax.experimental.pallas import tpu as pltpu


def _se_kernel(x_ref, w1t_ref, w2_ref, o_ref, gate_ref, *, chunk, inv_s):
    k = pl.program_id(1)

    @pl.when(k == 0)
    def _():
        x = x_ref[0].astype(jnp.float32)
        pool = jnp.sum(x, axis=-1, keepdims=True) * inv_s
        h = jax.lax.dot_general(
            pool, w1t_ref[...], (((0,), (0,)), ((), ())),
            preferred_element_type=jnp.float32)
        h = jnp.maximum(h, 0.0)
        logits = jax.lax.dot_general(
            w2_ref[...], h, (((1,), (1,)), ((), ())),
            preferred_element_type=jnp.float32)
        gate_ref[...] = jax.nn.sigmoid(logits)

    xc = x_ref[0, :, pl.ds(k * chunk, chunk)].astype(jnp.float32)
    o_ref[0] = (xc * gate_ref[...]).astype(o_ref.dtype)


def kernel(x, w1, w2):
    N, C, D, H, W = x.shape
    hid = w1.shape[0]
    S = D * H * W
    x2 = x.reshape(N, C, S)

    K = 1
    for cand in (4, 2):
        if S % (cand * 128) == 0 and (S // cand) * C * 4 >= 2 * 1024 * 1024:
            K = cand
            break
    chunk = S // K

    out = pl.pallas_call(
        functools.partial(_se_kernel, chunk=chunk, inv_s=1.0 / S),
        out_shape=jax.ShapeDtypeStruct((N, C, S), x.dtype),
        grid=(N, K),
        in_specs=[
            pl.BlockSpec((1, C, S), lambda n, k: (n, 0, 0)),
            pl.BlockSpec((C, hid), lambda n, k: (0, 0)),
            pl.BlockSpec((C, hid), lambda n, k: (0, 0)),
        ],
        out_specs=pl.BlockSpec((1, C, chunk), lambda n, k: (n, 0, k)),
        scratch_shapes=[pltpu.VMEM((C, 1), jnp.float32)],
        compiler_params=pltpu.CompilerParams(
            dimension_semantics=("parallel", "arbitrary"),
            vmem_limit_bytes=56 * 1024 * 1024,
        ),
    )(x2, jnp.transpose(w1), w2)

    return out.reshape(N, C, D, H, W)


# speedup vs baseline: 1.0727x; 1.0003x over previous
import functools

import jax
import jax.numpy as jnp
from jax.experimental import pallas as pl
from jax.experimental.pallas import tpu as pltpu


def _se_kernel(x_ref, w1t_ref, w2_ref, o_ref, gate_ref, *, chunk, inv_s):
    k = pl.program_id(1)

    @pl.when(k == 0)
    def _():
        x = x_ref[0].astype(jnp.float32)
        pool = jnp.sum(x, axis=-1, keepdims=True) * inv_s
        h = jax.lax.dot_general(
            pool, w1t_ref[...], (((0,), (0,)), ((), ())),
            preferred_element_type=jnp.float32)
        h = jnp.maximum(h, 0.0)
        logits = jax.lax.dot_general(
            w2_ref[...], h, (((1,), (1,)), ((), ())),
            preferred_element_type=jnp.float32)
        gate_ref[...] = jax.nn.sigmoid(logits)

    xc = x_ref[0, :, pl.ds(k * chunk, chunk)].astype(jnp.float32)
    o_ref[0] = (xc * gate_ref[...]).astype(o_ref.dtype)


def kernel(x, w1, w2):
    N, C, D, H, W = x.shape
    hid = w1.shape[0]
    S = D * H * W
    x2 = x.reshape(N, C, S)

    K = 1
    for cand in (2,):
        if S % (cand * 128) == 0 and (S // cand) * C * 4 >= 2 * 1024 * 1024:
            K = cand
            break
    chunk = S // K

    out = pl.pallas_call(
        functools.partial(_se_kernel, chunk=chunk, inv_s=1.0 / S),
        out_shape=jax.ShapeDtypeStruct((N, C, S), x.dtype),
        grid=(N, K),
        in_specs=[
            pl.BlockSpec((1, C, S), lambda n, k: (n, 0, 0)),
            pl.BlockSpec((C, hid), lambda n, k: (0, 0)),
            pl.BlockSpec((C, hid), lambda n, k: (0, 0)),
        ],
        out_specs=pl.BlockSpec((1, C, chunk), lambda n, k: (n, 0, k)),
        scratch_shapes=[pltpu.VMEM((C, 1), jnp.float32)],
        compiler_params=pltpu.CompilerParams(
            dimension_semantics=("parallel", "arbitrary"),
            vmem_limit_bytes=56 * 1024 * 1024,
        ),
    )(x2, jnp.transpose(w1), w2)

    return out.reshape(N, C, D, H, W)


# speedup vs baseline: 1.1909x; 1.1102x over previous
import functools

import jax
import jax.numpy as jnp
from jax.experimental import pallas as pl
from jax.experimental.pallas import tpu as pltpu


def _se_fused(x_ref, w1t_ref, w2_ref, o_ref, *, inv_s):
    x = x_ref[0].astype(jnp.float32)
    pool = jnp.sum(x, axis=-1, keepdims=True) * inv_s
    h = jax.lax.dot_general(
        pool, w1t_ref[...], (((0,), (0,)), ((), ())),
        preferred_element_type=jnp.float32)
    h = jnp.maximum(h, 0.0)
    logits = jax.lax.dot_general(
        w2_ref[...], h, (((1,), (1,)), ((), ())),
        preferred_element_type=jnp.float32)
    gate = jax.nn.sigmoid(logits)
    o_ref[0] = (x * gate).astype(o_ref.dtype)


def kernel(x, w1, w2):
    N, C, D, H, W = x.shape
    hid = w1.shape[0]
    S = D * H * W
    x2 = x.reshape(N, C, S)

    out = pl.pallas_call(
        functools.partial(_se_fused, inv_s=1.0 / S),
        out_shape=jax.ShapeDtypeStruct((N, C, S), x.dtype),
        grid=(N,),
        in_specs=[
            pl.BlockSpec((1, C, S), lambda n: (n, 0, 0)),
            pl.BlockSpec((C, hid), lambda n: (0, 0)),
            pl.BlockSpec((C, hid), lambda n: (0, 0)),
        ],
        out_specs=pl.BlockSpec((1, C, S), lambda n: (n, 0, 0)),
        compiler_params=pltpu.CompilerParams(
            dimension_semantics=("parallel",),
            vmem_limit_bytes=56 * 1024 * 1024,
        ),
    )(x2, jnp.transpose(w1), w2)

    return out.reshape(N, C, D, H, W)


# speedup vs baseline: 1.2010x; 1.0085x over previous
import functools

import jax
import jax.numpy as jnp
from jax.experimental import pallas as pl
from jax.experimental.pallas import tpu as pltpu


def _se_fused(x_ref, w1t_ref, w2_ref, o_ref, *, inv_s):
    x = x_ref[0]
    pool = jnp.sum(x, axis=-1, keepdims=True) * inv_s
    h = jnp.maximum(
        jnp.sum(w1t_ref[...] * pool, axis=0, keepdims=True), 0.0)
    logits = jnp.sum(w2_ref[...] * h, axis=1, keepdims=True)
    gate = jax.nn.sigmoid(logits)
    o_ref[0] = x * gate


def kernel(x, w1, w2):
    N, C, D, H, W = x.shape
    hid = w1.shape[0]
    S = D * H * W
    x2 = x.reshape(N, C, S)

    out = pl.pallas_call(
        functools.partial(_se_fused, inv_s=1.0 / S),
        out_shape=jax.ShapeDtypeStruct((N, C, S), x.dtype),
        grid=(N,),
        in_specs=[
            pl.BlockSpec((1, C, S), lambda n: (n, 0, 0)),
            pl.BlockSpec((C, hid), lambda n: (0, 0)),
            pl.BlockSpec((C, hid), lambda n: (0, 0)),
        ],
        out_specs=pl.BlockSpec((1, C, S), lambda n: (n, 0, 0)),
        compiler_params=pltpu.CompilerParams(
            dimension_semantics=("parallel",),
            vmem_limit_bytes=56 * 1024 * 1024,
        ),
    )(x2, jnp.transpose(w1), w2)

    return out.reshape(N, C, D, H, W)
